# baseline (speedup 1.0000x reference)
import numpy as np
import jax
import jax.numpy as jnp
from jax.experimental import pallas as pl
from jax.experimental.pallas import tpu as pltpu

_AUDIO_RATE = 22050.0 / 256.0
_SIZES = [x * _AUDIO_RATE for x in [2.23147392, 2.62519274, 3.74199546, 5.78800454, 8.02371882]]
_LEVEL_N = [4096, 2048, 1024, 512, 256]
_LOWER = np.concatenate([
    np.full(n, ([0.0] + _SIZES)[i], np.float32) for i, n in enumerate(_LEVEL_N)
])
_UPPER = np.concatenate([
    np.full(n, _SIZES[i], np.float32) for i, n in enumerate(_LEVEL_N)
])

_B, _G, _C = 16, 30, 8
_A = sum(_LEVEL_N)
_ROWS = _A // 128


def _focal_kernel(starts_ref, ends_ref, acls_ref, cid_ref, x_ref,
                  p_ref, lo_ref, up_ref, out_ref):
    b = pl.program_id(0)
    cid = cid_ref[0, 0]
    cidf = cid.astype(jnp.float32)

    p = p_ref[...]
    lo = lo_ref[...]
    up = up_ref[...]

    def body(g, acc):
        s = starts_ref[b, g]
        e = ends_ref[b, g]
        c = acls_ref[b, g]
        l = p - s
        r = e - p
        mn = jnp.minimum(l, r)
        mx = jnp.maximum(l, r)
        ok = (mn >= 0.0) & (mx >= lo) & (mx < up) & (c == cidf)
        return jnp.maximum(acc, jnp.where(ok, 1.0, 0.0))

    posf = jax.lax.fori_loop(0, _G, body,
                             jnp.zeros((_ROWS, 128), jnp.float32))

    pos8 = jnp.concatenate([posf] * _C, axis=0)
    ri = jax.lax.broadcasted_iota(jnp.int32, (_C * _ROWS, 128), 0)
    chmask = jnp.where((ri >= cid * _ROWS) & (ri < (cid + 1) * _ROWS),
                       1.0, 0.0)
    tf = pos8 * chmask

    x = x_ref[0]
    cls = jnp.clip(x, 1e-4, 1.0 - 1e-4)
    u = cls + tf * (1.0 - 2.0 * cls)
    af = 0.75 - 0.5 * tf
    loss = af * u * u * (-jnp.log(1.0 - u))

    total = jnp.sum(loss)
    npos = jnp.sum(tf)
    per_b = total / jnp.maximum(npos, 1.0)

    @pl.when(b == 0)
    def _():
        out_ref[0, 0] = 0.0

    out_ref[0, 0] += per_b / _B


def kernel(classifications, annotations, anchors0, anchors1, anchors2,
           anchors3, anchors4, class_id):
    B, A, C = classifications.shape
    xt = jnp.transpose(classifications, (0, 2, 1)).reshape(B, C * _ROWS, 128)
    starts = annotations[:, :, 0]
    ends = annotations[:, :, 1]
    acls = annotations[:, :, 2]
    cid = jnp.asarray(class_id, jnp.int32).reshape(1, 1)
    p = jnp.concatenate([anchors0, anchors1, anchors2, anchors3,
                         anchors4]).reshape(_ROWS, 128)
    lo = jnp.asarray(_LOWER).reshape(_ROWS, 128)
    up = jnp.asarray(_UPPER).reshape(_ROWS, 128)

    out = pl.pallas_call(
        _focal_kernel,
        grid=(B,),
        in_specs=[
            pl.BlockSpec(memory_space=pltpu.SMEM),
            pl.BlockSpec(memory_space=pltpu.SMEM),
            pl.BlockSpec(memory_space=pltpu.SMEM),
            pl.BlockSpec(memory_space=pltpu.SMEM),
            pl.BlockSpec((1, C * _ROWS, 128), lambda b: (b, 0, 0)),
            pl.BlockSpec((_ROWS, 128), lambda b: (0, 0)),
            pl.BlockSpec((_ROWS, 128), lambda b: (0, 0)),
            pl.BlockSpec((_ROWS, 128), lambda b: (0, 0)),
        ],
        out_specs=pl.BlockSpec(memory_space=pltpu.SMEM),
        out_shape=jax.ShapeDtypeStruct((1, 1), jnp.float32),
        compiler_params=pltpu.CompilerParams(
            dimension_semantics=("arbitrary",)),
    )(starts, ends, acls, cid, xt, p, lo, up)
    return out[0, 0]
